# Initial kernel scaffold; baseline (speedup 1.0000x reference)
import jax
import jax.numpy as jnp
from jax.experimental import pallas as pl


def kernel(melspecs, centroids):
    raise NotImplementedError("write your pallas kernel here")


# speedup vs baseline: 214.0187x; 214.0187x over previous
import jax
import jax.numpy as jnp
from jax.experimental import pallas as pl


def _vq_kernel(mel_ref, cent_ref, out_ref):
    x = mel_ref[...]
    c = cent_ref[...]
    k_total = c.shape[0]
    c0 = c[0:1, :]
    best_d = jnp.abs(x - c0)
    best_v = jnp.broadcast_to(c0, x.shape)
    for k in range(1, k_total):
        ck = c[k:k + 1, :]
        d = jnp.abs(x - ck)
        take = d < best_d
        best_v = jnp.where(take, jnp.broadcast_to(ck, x.shape), best_v)
        best_d = jnp.where(take, d, best_d)
    out_ref[...] = best_v


def kernel(melspecs, centroids):
    b, t, n_mels = melspecs.shape
    k = centroids.shape[1]
    rows = b * t
    x = melspecs.reshape(rows, n_mels)
    ct = centroids.T
    blk = 2048
    grid = (rows // blk,)
    out = pl.pallas_call(
        _vq_kernel,
        grid=grid,
        in_specs=[
            pl.BlockSpec((blk, n_mels), lambda i: (i, 0)),
            pl.BlockSpec((k, n_mels), lambda i: (0, 0)),
        ],
        out_specs=pl.BlockSpec((blk, n_mels), lambda i: (i, 0)),
        out_shape=jax.ShapeDtypeStruct((rows, n_mels), melspecs.dtype),
    )(x, ct)
    return out.reshape(b, t, n_mels)


# speedup vs baseline: 234.1993x; 1.0943x over previous
import jax
import jax.numpy as jnp
from jax import lax
from jax.experimental import pallas as pl
from jax.experimental.pallas import tpu as pltpu


def _vq_kernel(x_ref, c_ref, o_ref, mid_ref, delta_ref, base_ref):
    k, lanes = c_ref.shape

    @pl.when(pl.program_id(0) == 0)
    def _prep():
        c = c_ref[...]
        ci = c[:, None, :]
        cj = c[None, :, :]
        ii = lax.broadcasted_iota(jnp.int32, (k, k, 1), 0)
        jj = lax.broadcasted_iota(jnp.int32, (k, k, 1), 1)
        rank = jnp.sum(
            jnp.where((cj < ci) | ((cj == ci) & (jj < ii)), 1, 0),
            axis=1,
        )
        rr = lax.broadcasted_iota(jnp.int32, (k, k, 1), 0)
        oh = (rank[None, :, :] == rr).astype(c.dtype)
        srt = jnp.sum(oh * c[None, :, :], axis=1)
        nxt = jnp.concatenate([srt[1:], srt[k - 1:]], axis=0)
        mid_ref[...] = 0.5 * (srt + nxt)
        delta_ref[...] = nxt - srt
        base_ref[...] = srt[0:1]

    x = x_ref[...]
    acc = jnp.broadcast_to(base_ref[...], x.shape)
    for j in range(k):
        acc = acc + jnp.where(x > mid_ref[j:j + 1, :], delta_ref[j:j + 1, :],
                              jnp.zeros((), x.dtype))
    o_ref[...] = acc


def kernel(melspecs, centroids):
    b, t, n_mels = melspecs.shape
    k = centroids.shape[1]
    n = b * t * n_mels
    lanes = 640
    rows = n // lanes
    x = melspecs.reshape(rows, lanes)
    ct = jnp.tile(centroids.T, (1, lanes // n_mels))
    blk = 512
    grid = (rows // blk,)
    out = pl.pallas_call(
        _vq_kernel,
        grid=grid,
        in_specs=[
            pl.BlockSpec((blk, lanes), lambda i: (i, 0)),
            pl.BlockSpec((k, lanes), lambda i: (0, 0)),
        ],
        out_specs=pl.BlockSpec((blk, lanes), lambda i: (i, 0)),
        out_shape=jax.ShapeDtypeStruct((rows, lanes), melspecs.dtype),
        scratch_shapes=[
            pltpu.VMEM((k, lanes), melspecs.dtype),
            pltpu.VMEM((k, lanes), melspecs.dtype),
            pltpu.VMEM((1, lanes), melspecs.dtype),
        ],
    )(x, ct)
    return out.reshape(b, t, n_mels)


# speedup vs baseline: 279.5753x; 1.1937x over previous
import jax
import jax.numpy as jnp
from jax import lax
from jax.experimental import pallas as pl
from jax.experimental.pallas import tpu as pltpu


def _vq_kernel(x_ref, c_ref, o_ref, mid_ref, delta_ref, base_ref):
    k, lanes = c_ref.shape

    @pl.when(pl.program_id(0) == 0)
    def _prep():
        c = c_ref[...]
        ci = c[:, None, :]
        cj = c[None, :, :]
        ii = lax.broadcasted_iota(jnp.int32, (k, k, 1), 0)
        jj = lax.broadcasted_iota(jnp.int32, (k, k, 1), 1)
        rank = jnp.sum(
            jnp.where((cj < ci) | ((cj == ci) & (jj < ii)), 1, 0),
            axis=1,
        )
        rr = lax.broadcasted_iota(jnp.int32, (k, k, 1), 0)
        oh = (rank[None, :, :] == rr).astype(c.dtype)
        srt = jnp.sum(oh * c[None, :, :], axis=1)
        nxt = jnp.concatenate([srt[1:], srt[k - 1:]], axis=0)
        mid_ref[...] = 0.5 * (srt + nxt)
        delta_ref[...] = nxt - srt
        base_ref[...] = srt[0:1]

    x = x_ref[...]
    acc = jnp.broadcast_to(base_ref[...], x.shape)
    for j in range(k):
        acc = acc + jnp.where(x > mid_ref[j:j + 1, :], delta_ref[j:j + 1, :],
                              jnp.zeros((), x.dtype))
    o_ref[...] = acc


def kernel(melspecs, centroids):
    b, t, n_mels = melspecs.shape
    k = centroids.shape[1]
    rows = b * t
    lanes = n_mels
    x = melspecs.reshape(rows, lanes)
    ct = centroids.T
    blk = 2048
    grid = (rows // blk,)
    out = pl.pallas_call(
        _vq_kernel,
        grid=grid,
        in_specs=[
            pl.BlockSpec((blk, lanes), lambda i: (i, 0)),
            pl.BlockSpec((k, lanes), lambda i: (0, 0)),
        ],
        out_specs=pl.BlockSpec((blk, lanes), lambda i: (i, 0)),
        out_shape=jax.ShapeDtypeStruct((rows, lanes), melspecs.dtype),
        scratch_shapes=[
            pltpu.VMEM((k, lanes), melspecs.dtype),
            pltpu.VMEM((k, lanes), melspecs.dtype),
            pltpu.VMEM((1, lanes), melspecs.dtype),
        ],
    )(x, ct)
    return out.reshape(b, t, n_mels)


# speedup vs baseline: 318.1372x; 1.1379x over previous
import jax
import jax.numpy as jnp
from jax import lax
from jax.experimental import pallas as pl
from jax.experimental.pallas import tpu as pltpu


def _vq_kernel(x_ref, c_ref, o_ref, mid_ref, delta_ref, base_ref):
    k, lanes = c_ref.shape

    @pl.when((pl.program_id(0) == 0) & (pl.program_id(1) == 0))
    def _prep():
        c = c_ref[...]
        ci = c[:, None, :]
        cj = c[None, :, :]
        ii = lax.broadcasted_iota(jnp.int32, (k, k, 1), 0)
        jj = lax.broadcasted_iota(jnp.int32, (k, k, 1), 1)
        rank = jnp.sum(
            jnp.where((cj < ci) | ((cj == ci) & (jj < ii)), 1, 0),
            axis=1,
        )
        rr = lax.broadcasted_iota(jnp.int32, (k, k, 1), 0)
        oh = (rank[None, :, :] == rr).astype(c.dtype)
        srt = jnp.sum(oh * c[None, :, :], axis=1)
        nxt = jnp.concatenate([srt[1:], srt[k - 1:]], axis=0)
        mid_ref[...] = 0.5 * (srt + nxt)
        delta_ref[...] = nxt - srt
        base_ref[...] = srt[0:1]

    x = x_ref[0]
    acc = jnp.broadcast_to(base_ref[...], x.shape)
    for j in range(k):
        acc = acc + jnp.where(x > mid_ref[j:j + 1, :], delta_ref[j:j + 1, :],
                              jnp.zeros((), x.dtype))
    o_ref[0] = acc


def kernel(melspecs, centroids):
    b, t, n_mels = melspecs.shape
    k = centroids.shape[1]
    ct = centroids.T
    blk_t = 2048
    grid = (b, t // blk_t)
    out = pl.pallas_call(
        _vq_kernel,
        grid=grid,
        in_specs=[
            pl.BlockSpec((1, blk_t, n_mels), lambda i, j: (i, j, 0)),
            pl.BlockSpec((k, n_mels), lambda i, j: (0, 0)),
        ],
        out_specs=pl.BlockSpec((1, blk_t, n_mels), lambda i, j: (i, j, 0)),
        out_shape=jax.ShapeDtypeStruct((b, t, n_mels), melspecs.dtype),
        scratch_shapes=[
            pltpu.VMEM((k, n_mels), melspecs.dtype),
            pltpu.VMEM((k, n_mels), melspecs.dtype),
            pltpu.VMEM((1, n_mels), melspecs.dtype),
        ],
    )(melspecs, ct)
    return out


# speedup vs baseline: 378.9831x; 1.1913x over previous
import jax
import jax.numpy as jnp
from jax import lax
from jax.experimental import pallas as pl
from jax.experimental.pallas import tpu as pltpu


def _tree_select(cands, bits):
    vals = list(cands)
    for b in reversed(bits):
        vals = [jnp.where(b, vals[2 * i + 1], vals[2 * i])
                for i in range(len(vals) // 2)]
    return vals[0]


def _vq_kernel(x_ref, c_ref, o_ref, srt_ref, mid_ref):
    k, lanes = c_ref.shape

    @pl.when((pl.program_id(0) == 0) & (pl.program_id(1) == 0))
    def _prep():
        c = c_ref[...]
        ci = c[:, None, :]
        cj = c[None, :, :]
        ii = lax.broadcasted_iota(jnp.int32, (k, k, 1), 0)
        jj = lax.broadcasted_iota(jnp.int32, (k, k, 1), 1)
        rank = jnp.sum(
            jnp.where((cj < ci) | ((cj == ci) & (jj < ii)), 1, 0),
            axis=1,
        )
        rr = lax.broadcasted_iota(jnp.int32, (k, k, 1), 0)
        oh = (rank[None, :, :] == rr).astype(c.dtype)
        srt = jnp.sum(oh * c[None, :, :], axis=1)
        nxt = jnp.concatenate([srt[1:], srt[k - 1:]], axis=0)
        srt_ref[...] = srt
        mid_ref[...] = 0.5 * (srt + nxt)

    def m(j):
        return mid_ref[j:j + 1, :]

    x = x_ref[0]
    levels = k.bit_length() - 1
    bits = []
    for l in range(levels):
        step = 1 << (levels - 1 - l)
        cands = [m(p * 2 * step + step - 1) for p in range(1 << l)]
        boundary = _tree_select(cands, bits)
        bits.append(x > boundary)
    vals = [srt_ref[j:j + 1, :] for j in range(k)]
    o_ref[0] = _tree_select(vals, bits)


def kernel(melspecs, centroids):
    b, t, n_mels = melspecs.shape
    k = centroids.shape[1]
    ct = centroids.T
    blk_t = 2048
    grid = (b, t // blk_t)
    out = pl.pallas_call(
        _vq_kernel,
        grid=grid,
        in_specs=[
            pl.BlockSpec((1, blk_t, n_mels), lambda i, j: (i, j, 0)),
            pl.BlockSpec((k, n_mels), lambda i, j: (0, 0)),
        ],
        out_specs=pl.BlockSpec((1, blk_t, n_mels), lambda i, j: (i, j, 0)),
        out_shape=jax.ShapeDtypeStruct((b, t, n_mels), melspecs.dtype),
        scratch_shapes=[
            pltpu.VMEM((k, n_mels), melspecs.dtype),
            pltpu.VMEM((k, n_mels), melspecs.dtype),
        ],
    )(melspecs, ct)
    return out
